# baseline (speedup 1.0000x reference)
import functools

import jax
import jax.numpy as jnp
from jax import lax
from jax.experimental import pallas as pl
from jax.experimental.pallas import tpu as pltpu
from jax.experimental.pallas import tpu_sc as plsc

_NUM_FIELDS = 13
_VOCAB = 100000
_EMB = 64
_BATCH = 16384
_CAT = _NUM_FIELDS * _EMB
_H0, _H1, _OUT = 512, 256, 128
_TOTAL_ROWS = _BATCH * _NUM_FIELDS

_NC, _NS = 2, 16
_NW = _NC * _NS
_PPW = _TOTAL_ROWS // _NW
_NGRP = _PPW // 16
_GPF = 8

_BW = 256
_BPF = (_VOCAB + _BW - 1) // _BW
_PART_W = _VOCAB - (_BPF - 1) * _BW
_QSTRIDE = _BPF * _BW
_NBLK = _NUM_FIELDS * _BPF
_NRING = 4

_sc_mesh = plsc.VectorSubcoreMesh(core_axis_name="c", subcore_axis_name="s")

_BIG = 2 ** 30


@functools.partial(
    pl.kernel,
    out_type=jax.ShapeDtypeStruct((_TOTAL_ROWS * _EMB,), jnp.float32),
    mesh=_sc_mesh,
    scratch_types=[
        pltpu.VMEM((_PPW,), jnp.int32),
        pltpu.VMEM((_NRING, _EMB, _BW), jnp.float32),
        pltpu.VMEM((_NRING, _EMB, 32), jnp.float32),
        pltpu.VMEM((_GPF * 16 * _EMB,), jnp.float32),
        [pltpu.SemaphoreType.DMA] * _NRING,
        pltpu.SemaphoreType.DMA,
    ],
    compiler_params=pltpu.CompilerParams(
        use_tc_tiling_on_sc=True, needs_layout_passes=False),
)
def _sc_scan_gather(tt_hbm, q_hbm, out_hbm, q_v, blk_v, pblk_v, rows_v,
                    sems, fsem):
    wid = lax.axis_index("s") * _NC + lax.axis_index("c")
    base = wid * _PPW
    pltpu.sync_copy(q_hbm.at[pl.ds(base, _PPW)], q_v)

    def _emit(b, do_wait):
        c = lax.min(b, jnp.int32(_NBLK - 1))
        f = c // _BPF
        tf = c - f * _BPF
        full = tf < _BPF - 1

        def go(src, dst, j):
            if do_wait:
                pltpu.make_async_copy(src, dst, sems[j]).wait()
            else:
                pltpu.async_copy(src, dst, sems[j])

        for j in range(_NRING):
            @pl.when(lax.rem(b, _NRING) == j)
            def _(j=j):
                @pl.when(full)
                def _():
                    go(tt_hbm.at[f, :, pl.ds(tf * _BW, 128)],
                       blk_v.at[j, :, pl.ds(0, 128)], j)
                    go(tt_hbm.at[f, :, pl.ds(tf * _BW + 128, 128)],
                       blk_v.at[j, :, pl.ds(128, 128)], j)

                @pl.when(jnp.logical_not(full))
                def _():
                    go(tt_hbm.at[f, :, pl.ds((_BPF - 1) * _BW, 128)],
                       blk_v.at[j, :, pl.ds(0, 128)], j)
                    go(tt_hbm.at[f, :, pl.ds((_BPF - 1) * _BW + 128,
                                             _PART_W - 128)],
                       pblk_v.at[j], j)

    def stage(b):
        _emit(b, False)

    def wait_blk(b):
        _emit(b, True)

    q0 = q_v[pl.ds(0, 16)]
    cur0 = lax.reduce_min(q0 >> 8, (0,))
    for k in range(_NRING):
        stage(cur0 + k)
    wait_blk(cur0)
    init = (cur0, cur0 + _NRING - 1, cur0)

    def group_body(g, carry):
        qg = q_v[pl.ds(g * 16, 16)]
        blocks = qg >> 8
        cols = qg & (_BW - 1)
        rowbase = (lax.rem(g, _GPF) * 16) * _EMB + lax.iota(jnp.int32, 16) * _EMB

        def run_body(state):
            cur, hi_i, hi_w, rem = state
            t = lax.reduce_min(jnp.where(rem, blocks, _BIG), (0,))

            def advance(args):
                cur, hi_i, hi_w = args

                def adv_cond(st):
                    i, w = st
                    return jnp.logical_or(i < t + _NRING - 1, w < t)

                def adv_body(st):
                    i, w = st
                    can_issue = jnp.logical_and(i < t + _NRING - 1,
                                                i - (_NRING - 1) <= w)
                    @pl.when(can_issue)
                    def _():
                        stage(i + 1)

                    @pl.when(jnp.logical_not(can_issue))
                    def _():
                        wait_blk(w + 1)

                    return (jnp.where(can_issue, i + 1, i),
                            jnp.where(can_issue, w, w + 1))

                hi_i, hi_w = lax.while_loop(adv_cond, adv_body, (hi_i, hi_w))
                return t, hi_i, hi_w

            cur, hi_i, hi_w = lax.cond(
                t != cur, advance, lambda a: a, (cur, hi_i, hi_w))

            m = rem & (blocks == t)
            partial = (t - (t // _BPF) * _BPF) == (_BPF - 1)
            m_tail = m & partial & (cols >= 128)
            m_main = m & jnp.logical_not(m_tail)
            sidx = jnp.full((16,), lax.rem(t, _NRING), dtype=jnp.int32)
            for e in range(_EMB):
                eidx = jnp.full((16,), e, dtype=jnp.int32)
                vals = plsc.load_gather(blk_v, [sidx, eidx, cols])
                plsc.store_scatter(rows_v, [rowbase + e], vals, mask=m_main)

            @pl.when(jnp.any(m_tail))
            def _():
                cols_t = jnp.where(m_tail, cols - 128, 0)
                for e in range(_EMB):
                    eidx = jnp.full((16,), e, dtype=jnp.int32)
                    vals = plsc.load_gather(pblk_v, [sidx, eidx, cols_t])
                    plsc.store_scatter(rows_v, [rowbase + e], vals, mask=m_tail)

            return cur, hi_i, hi_w, rem & jnp.logical_not(m)

        def run_cond(state):
            return jnp.any(state[3])

        cur, hi_i, hi_w, _ = lax.while_loop(
            run_cond, run_body,
            carry + (jnp.ones((16,), dtype=jnp.bool_),))

        @pl.when(lax.rem(g, _GPF) == _GPF - 1)
        def _():
            pltpu.async_copy(
                rows_v,
                out_hbm.at[pl.ds((base + (g - (_GPF - 1)) * 16) * _EMB,
                                 _GPF * 16 * _EMB)],
                fsem,
            ).wait()

        return cur, hi_i, hi_w

    cur, hi_i, hi_w = lax.fori_loop(0, _NGRP, group_body, init)

    def drain_body(st):
        i, w = st
        wait_blk(w + 1)
        return i, w + 1

    lax.while_loop(lambda st: st[1] < st[0], drain_body, (hi_i, hi_w))


_CH = 128
_NCH = _PPW // _CH


@functools.partial(
    pl.kernel,
    out_type=jax.ShapeDtypeStruct((_TOTAL_ROWS, _EMB), jnp.float32),
    mesh=_sc_mesh,
    scratch_types=[
        pltpu.VMEM((_NCH, _CH), jnp.int32),
        pltpu.VMEM((2, _CH, _EMB), jnp.float32),
        pltpu.SemaphoreType.DMA,
        pltpu.SemaphoreType.DMA,
    ],
    compiler_params=pltpu.CompilerParams(use_tc_tiling_on_sc=False),
)
def _sc_unpermute(src_hbm, rid_hbm, out_hbm, rid_v, rows_v, lsem, wsem):
    wid = lax.axis_index("s") * _NC + lax.axis_index("c")
    pltpu.sync_copy(rid_hbm.at[wid], rid_v)
    base = wid * _NCH

    pltpu.async_copy(src_hbm.at[pl.ds(base * _CH, _CH)], rows_v.at[0], lsem)

    def body(j, carry):
        slot = lax.rem(j, 2)
        nslot = 1 - slot

        @pl.when(j + 1 < _NCH)
        def _():
            pltpu.async_copy(
                src_hbm.at[pl.ds((base + j + 1) * _CH, _CH)],
                rows_v.at[nslot], lsem)

        pltpu.make_async_copy(
            src_hbm.at[pl.ds((base + j) * _CH, _CH)], rows_v.at[slot], lsem
        ).wait()
        pltpu.async_copy(rows_v.at[slot], out_hbm.at[rid_v.at[j]], wsem).wait()
        return carry

    lax.fori_loop(0, _NCH, body, 0)


_BM = 1024


def _mlp_body(x_ref, w1_ref, b1_ref, w2_ref, b2_ref, w3_ref, b3_ref, o_ref):
    h = jnp.dot(x_ref[...].astype(jnp.bfloat16), w1_ref[...].astype(jnp.bfloat16),
                preferred_element_type=jnp.float32)
    h = jnp.maximum(h + b1_ref[...], 0.0).astype(jnp.bfloat16)
    h = jnp.dot(h, w2_ref[...].astype(jnp.bfloat16),
                preferred_element_type=jnp.float32)
    h = jnp.maximum(h + b2_ref[...], 0.0).astype(jnp.bfloat16)
    o_ref[...] = jnp.dot(h, w3_ref[...].astype(jnp.bfloat16),
                         preferred_element_type=jnp.float32) + b3_ref[...]


def _tc_mlp(x, W1, b1, W2, b2, W3, b3):
    nb = _BATCH // _BM
    return pl.pallas_call(
        _mlp_body,
        grid=(nb,),
        in_specs=[
            pl.BlockSpec((_BM, _CAT), lambda i: (i, 0)),
            pl.BlockSpec((_CAT, _H0), lambda i: (0, 0)),
            pl.BlockSpec((1, _H0), lambda i: (0, 0)),
            pl.BlockSpec((_H0, _H1), lambda i: (0, 0)),
            pl.BlockSpec((1, _H1), lambda i: (0, 0)),
            pl.BlockSpec((_H1, _OUT), lambda i: (0, 0)),
            pl.BlockSpec((1, _OUT), lambda i: (0, 0)),
        ],
        out_specs=pl.BlockSpec((_BM, _OUT), lambda i: (i, 0)),
        out_shape=jax.ShapeDtypeStruct((_BATCH, _OUT), jnp.float32),
    )(x, W1, b1.reshape(1, _H0), W2, b2.reshape(1, _H1), W3, b3.reshape(1, _OUT))


def _tower_gather(emb, field_idx):
    off = jnp.arange(_NUM_FIELDS, dtype=jnp.int32) * _QSTRIDE
    q = (field_idx.astype(jnp.int32) + off[None, :]).reshape(-1)
    rid = lax.iota(jnp.int32, _TOTAL_ROWS)
    q_sorted, rid_sorted = lax.sort([q, rid], num_keys=1)
    tt = jnp.transpose(emb, (0, 2, 1))
    flat = _sc_scan_gather(tt, q_sorted)
    src = flat.reshape(_TOTAL_ROWS, _EMB)
    out = _sc_unpermute(src, rid_sorted.reshape(_NW, _NCH, _CH))
    return out.reshape(_BATCH, _CAT)


def kernel(user_input, item_input, user_emb, item_emb, W_user, b_user,
           W_item, b_item, W2, b2, W3, b3):
    u_cat = _tower_gather(user_emb, user_input)
    out1 = _tc_mlp(u_cat, W_user, b_user, W2, b2, W3, b3)
    v_cat = _tower_gather(item_emb, item_input)
    out2 = _tc_mlp(v_cat, W_item, b_item, W2, b2, W3, b3)
    return (out1, out2)


# speedup vs baseline: 1.0242x; 1.0242x over previous
import functools

import jax
import jax.numpy as jnp
from jax import lax
from jax.experimental import pallas as pl
from jax.experimental.pallas import tpu as pltpu
from jax.experimental.pallas import tpu_sc as plsc

_NUM_FIELDS = 13
_VOCAB = 100000
_EMB = 64
_BATCH = 16384
_CAT = _NUM_FIELDS * _EMB
_H0, _H1, _OUT = 512, 256, 128
_TOTAL_ROWS = _BATCH * _NUM_FIELDS

_NC, _NS = 2, 16
_NW = _NC * _NS
_PPW = _TOTAL_ROWS // _NW
_NGRP = _PPW // 16
_GPF = 8

_BW = 768
_BPF = (_VOCAB + _BW - 1) // _BW
_PART_W = _VOCAB - (_BPF - 1) * _BW
_QSTRIDE = _BPF * _BW
_NBLK = _NUM_FIELDS * _BPF
_NRING = 2

_sc_mesh = plsc.VectorSubcoreMesh(core_axis_name="c", subcore_axis_name="s")

_BIG = 2 ** 30


@functools.partial(
    pl.kernel,
    out_type=jax.ShapeDtypeStruct((_TOTAL_ROWS * _EMB,), jnp.float32),
    mesh=_sc_mesh,
    scratch_types=[
        pltpu.VMEM((_PPW,), jnp.int32),
        pltpu.VMEM((_NRING, _EMB, _BW), jnp.float32),
        pltpu.VMEM((_NRING, _EMB, 32), jnp.float32),
        pltpu.VMEM((_GPF * 16 * _EMB,), jnp.float32),
        [pltpu.SemaphoreType.DMA] * _NRING,
        pltpu.SemaphoreType.DMA,
    ],
    compiler_params=pltpu.CompilerParams(
        use_tc_tiling_on_sc=True, needs_layout_passes=False),
)
def _sc_scan_gather(tt_hbm, q_hbm, out_hbm, q_v, blk_v, pblk_v, rows_v,
                    sems, fsem):
    wid = lax.axis_index("s") * _NC + lax.axis_index("c")
    base = wid * _PPW
    pltpu.sync_copy(q_hbm.at[pl.ds(base, _PPW)], q_v)

    def _emit(b, do_wait):
        c = lax.min(b, jnp.int32(_NBLK - 1))
        f = c // _BPF
        tf = c - f * _BPF
        full = tf < _BPF - 1

        def go(src, dst, j):
            if do_wait:
                pltpu.make_async_copy(src, dst, sems[j]).wait()
            else:
                pltpu.async_copy(src, dst, sems[j])

        for j in range(_NRING):
            @pl.when(lax.rem(b, _NRING) == j)
            def _(j=j):
                @pl.when(full)
                def _():
                    go(tt_hbm.at[f, :, pl.ds(tf * _BW, _BW)],
                       blk_v.at[j], j)

                @pl.when(jnp.logical_not(full))
                def _():
                    go(tt_hbm.at[f, :, pl.ds((_BPF - 1) * _BW, 128)],
                       blk_v.at[j, :, pl.ds(0, 128)], j)
                    go(tt_hbm.at[f, :, pl.ds((_BPF - 1) * _BW + 128,
                                             _PART_W - 128)],
                       pblk_v.at[j], j)

    def stage(b):
        _emit(b, False)

    def wait_blk(b):
        _emit(b, True)

    q0 = q_v[pl.ds(0, 16)]
    cur0 = lax.reduce_min((q0 >> 8) // 3, (0,))
    for k in range(_NRING):
        stage(cur0 + k)
    wait_blk(cur0)
    init = (cur0, cur0 + _NRING - 1, cur0)

    def group_body(g, carry):
        qg = q_v[pl.ds(g * 16, 16)]
        blocks = (qg >> 8) // 3
        cols = qg - blocks * _BW
        rowbase = (lax.rem(g, _GPF) * 16) * _EMB + lax.iota(jnp.int32, 16) * _EMB

        def run_body(state):
            cur, hi_i, hi_w, rem = state
            t = lax.reduce_min(jnp.where(rem, blocks, _BIG), (0,))

            def advance(args):
                cur, hi_i, hi_w = args

                def adv_cond(st):
                    i, w = st
                    return jnp.logical_or(i < t + _NRING - 1, w < t)

                def adv_body(st):
                    i, w = st
                    can_issue = jnp.logical_and(i < t + _NRING - 1,
                                                i - (_NRING - 1) <= w)
                    @pl.when(can_issue)
                    def _():
                        stage(i + 1)

                    @pl.when(jnp.logical_not(can_issue))
                    def _():
                        wait_blk(w + 1)

                    return (jnp.where(can_issue, i + 1, i),
                            jnp.where(can_issue, w, w + 1))

                hi_i, hi_w = lax.while_loop(adv_cond, adv_body, (hi_i, hi_w))
                return t, hi_i, hi_w

            cur, hi_i, hi_w = lax.cond(
                t != cur, advance, lambda a: a, (cur, hi_i, hi_w))

            m = rem & (blocks == t)
            partial = (t - (t // _BPF) * _BPF) == (_BPF - 1)
            m_tail = m & partial & (cols >= 128)
            m_main = m & jnp.logical_not(m_tail)
            sidx = jnp.full((16,), lax.rem(t, _NRING), dtype=jnp.int32)
            for e in range(_EMB):
                eidx = jnp.full((16,), e, dtype=jnp.int32)
                vals = plsc.load_gather(blk_v, [sidx, eidx, cols])
                plsc.store_scatter(rows_v, [rowbase + e], vals, mask=m_main)

            @pl.when(jnp.any(m_tail))
            def _():
                cols_t = jnp.where(m_tail, cols - 128, 0)
                for e in range(_EMB):
                    eidx = jnp.full((16,), e, dtype=jnp.int32)
                    vals = plsc.load_gather(pblk_v, [sidx, eidx, cols_t])
                    plsc.store_scatter(rows_v, [rowbase + e], vals, mask=m_tail)

            return cur, hi_i, hi_w, rem & jnp.logical_not(m)

        def run_cond(state):
            return jnp.any(state[3])

        cur, hi_i, hi_w, _ = lax.while_loop(
            run_cond, run_body,
            carry + (jnp.ones((16,), dtype=jnp.bool_),))

        @pl.when(lax.rem(g, _GPF) == _GPF - 1)
        def _():
            pltpu.async_copy(
                rows_v,
                out_hbm.at[pl.ds((base + (g - (_GPF - 1)) * 16) * _EMB,
                                 _GPF * 16 * _EMB)],
                fsem,
            ).wait()

        return cur, hi_i, hi_w

    cur, hi_i, hi_w = lax.fori_loop(0, _NGRP, group_body, init)

    def drain_body(st):
        i, w = st
        wait_blk(w + 1)
        return i, w + 1

    lax.while_loop(lambda st: st[1] < st[0], drain_body, (hi_i, hi_w))


_CH = 128
_NCH = _PPW // _CH


@functools.partial(
    pl.kernel,
    out_type=jax.ShapeDtypeStruct((_TOTAL_ROWS, _EMB), jnp.float32),
    mesh=_sc_mesh,
    scratch_types=[
        pltpu.VMEM((_NCH, _CH), jnp.int32),
        pltpu.VMEM((2, _CH, _EMB), jnp.float32),
        pltpu.SemaphoreType.DMA,
        pltpu.SemaphoreType.DMA,
    ],
    compiler_params=pltpu.CompilerParams(use_tc_tiling_on_sc=False),
)
def _sc_unpermute(src_hbm, rid_hbm, out_hbm, rid_v, rows_v, lsem, wsem):
    wid = lax.axis_index("s") * _NC + lax.axis_index("c")
    pltpu.sync_copy(rid_hbm.at[wid], rid_v)
    base = wid * _NCH

    pltpu.async_copy(src_hbm.at[pl.ds(base * _CH, _CH)], rows_v.at[0], lsem)

    def body(j, carry):
        slot = lax.rem(j, 2)
        nslot = 1 - slot

        @pl.when(j + 1 < _NCH)
        def _():
            pltpu.async_copy(
                src_hbm.at[pl.ds((base + j + 1) * _CH, _CH)],
                rows_v.at[nslot], lsem)

        pltpu.make_async_copy(
            src_hbm.at[pl.ds((base + j) * _CH, _CH)], rows_v.at[slot], lsem
        ).wait()
        pltpu.async_copy(rows_v.at[slot], out_hbm.at[rid_v.at[j]], wsem).wait()
        return carry

    lax.fori_loop(0, _NCH, body, 0)


_BM = 1024


def _mlp_body(x_ref, w1_ref, b1_ref, w2_ref, b2_ref, w3_ref, b3_ref, o_ref):
    h = jnp.dot(x_ref[...].astype(jnp.bfloat16), w1_ref[...].astype(jnp.bfloat16),
                preferred_element_type=jnp.float32)
    h = jnp.maximum(h + b1_ref[...], 0.0).astype(jnp.bfloat16)
    h = jnp.dot(h, w2_ref[...].astype(jnp.bfloat16),
                preferred_element_type=jnp.float32)
    h = jnp.maximum(h + b2_ref[...], 0.0).astype(jnp.bfloat16)
    o_ref[...] = jnp.dot(h, w3_ref[...].astype(jnp.bfloat16),
                         preferred_element_type=jnp.float32) + b3_ref[...]


def _tc_mlp(x, W1, b1, W2, b2, W3, b3):
    nb = _BATCH // _BM
    return pl.pallas_call(
        _mlp_body,
        grid=(nb,),
        in_specs=[
            pl.BlockSpec((_BM, _CAT), lambda i: (i, 0)),
            pl.BlockSpec((_CAT, _H0), lambda i: (0, 0)),
            pl.BlockSpec((1, _H0), lambda i: (0, 0)),
            pl.BlockSpec((_H0, _H1), lambda i: (0, 0)),
            pl.BlockSpec((1, _H1), lambda i: (0, 0)),
            pl.BlockSpec((_H1, _OUT), lambda i: (0, 0)),
            pl.BlockSpec((1, _OUT), lambda i: (0, 0)),
        ],
        out_specs=pl.BlockSpec((_BM, _OUT), lambda i: (i, 0)),
        out_shape=jax.ShapeDtypeStruct((_BATCH, _OUT), jnp.float32),
    )(x, W1, b1.reshape(1, _H0), W2, b2.reshape(1, _H1), W3, b3.reshape(1, _OUT))


def _tower_gather(emb, field_idx):
    off = jnp.arange(_NUM_FIELDS, dtype=jnp.int32) * _QSTRIDE
    q = (field_idx.astype(jnp.int32) + off[None, :]).reshape(-1)
    rid = lax.iota(jnp.int32, _TOTAL_ROWS)
    q_sorted, rid_sorted = lax.sort([q, rid], num_keys=1)
    tt = jnp.transpose(emb, (0, 2, 1))
    flat = _sc_scan_gather(tt, q_sorted)
    src = flat.reshape(_TOTAL_ROWS, _EMB)
    out = _sc_unpermute(src, rid_sorted.reshape(_NW, _NCH, _CH))
    return out.reshape(_BATCH, _CAT)


def kernel(user_input, item_input, user_emb, item_emb, W_user, b_user,
           W_item, b_item, W2, b2, W3, b3):
    u_cat = _tower_gather(user_emb, user_input)
    out1 = _tc_mlp(u_cat, W_user, b_user, W2, b2, W3, b3)
    v_cat = _tower_gather(item_emb, item_input)
    out2 = _tc_mlp(v_cat, W_item, b_item, W2, b2, W3, b3)
    return (out1, out2)


# speedup vs baseline: 1.3233x; 1.2921x over previous
import functools

import jax
import jax.numpy as jnp
from jax import lax
from jax.experimental import pallas as pl
from jax.experimental.pallas import tpu as pltpu
from jax.experimental.pallas import tpu_sc as plsc

_NUM_FIELDS = 13
_VOCAB = 100000
_EMB = 64
_BATCH = 16384
_CAT = _NUM_FIELDS * _EMB
_H0, _H1, _OUT = 512, 256, 128
_TOTAL_ROWS = _BATCH * _NUM_FIELDS

_NC, _NS = 2, 16
_NW = _NC * _NS
_PPW = _TOTAL_ROWS // _NW
_NGRP = _PPW // 16
_GPF = 8

_BW = 512
_BPF = (_VOCAB + _BW - 1) // _BW
_PART_W = _VOCAB - (_BPF - 1) * _BW
_QSTRIDE = _BPF * _BW
_NBLK = _NUM_FIELDS * _BPF
_PITCH = _EMB + 1

_sc_mesh = plsc.VectorSubcoreMesh(core_axis_name="c", subcore_axis_name="s")

_BIG = 2 ** 30


@functools.partial(
    pl.kernel,
    out_type=jax.ShapeDtypeStruct((_TOTAL_ROWS * _PITCH,), jnp.float32),
    mesh=_sc_mesh,
    scratch_types=[
        pltpu.VMEM((_PPW,), jnp.int32),
        pltpu.VMEM((2, _EMB, _BW), jnp.float32),
        pltpu.VMEM((2, _EMB, 32), jnp.float32),
        pltpu.VMEM((_GPF * 16 * _PITCH,), jnp.float32),
        pltpu.SemaphoreType.DMA,
        pltpu.SemaphoreType.DMA,
        pltpu.SemaphoreType.DMA,
    ],
    compiler_params=pltpu.CompilerParams(
        use_tc_tiling_on_sc=True, needs_layout_passes=False),
)
def _sc_scan_gather(tt_hbm, q_hbm, out_hbm, q_v, blk_v, pblk_v, rows_v,
                    ssem, psem, fsem):
    wid = lax.axis_index("s") * _NC + lax.axis_index("c")
    base = wid * _PPW
    pltpu.sync_copy(q_hbm.at[pl.ds(base, _PPW)], q_v)

    def stage(blk, slot, sem):
        f = blk // _BPF
        tf = blk - f * _BPF

        @pl.when(tf < _BPF - 1)
        def _():
            pltpu.async_copy(
                tt_hbm.at[f, :, pl.ds(tf * _BW, _BW)], blk_v.at[slot], sem)

        @pl.when(tf == _BPF - 1)
        def _():
            pltpu.async_copy(
                tt_hbm.at[f, :, pl.ds((_BPF - 1) * _BW, 128)],
                blk_v.at[slot, :, pl.ds(0, 128)], sem)
            pltpu.async_copy(
                tt_hbm.at[f, :, pl.ds((_BPF - 1) * _BW + 128, _PART_W - 128)],
                pblk_v.at[slot], sem)

    def stage_wait(blk, slot, sem):
        f = blk // _BPF
        tf = blk - f * _BPF

        @pl.when(tf < _BPF - 1)
        def _():
            pltpu.make_async_copy(
                tt_hbm.at[f, :, pl.ds(tf * _BW, _BW)], blk_v.at[slot], sem
            ).wait()

        @pl.when(tf == _BPF - 1)
        def _():
            pltpu.make_async_copy(
                tt_hbm.at[f, :, pl.ds((_BPF - 1) * _BW, 128)],
                blk_v.at[slot, :, pl.ds(0, 128)], sem
            ).wait()
            pltpu.make_async_copy(
                tt_hbm.at[f, :, pl.ds((_BPF - 1) * _BW + 128, _PART_W - 128)],
                pblk_v.at[slot], sem
            ).wait()

    q0 = q_v[pl.ds(0, 16)]
    cur0 = lax.reduce_min(q0 >> 9, (0,))
    stage(cur0, 0, ssem)
    stage_wait(cur0, 0, ssem)
    nxt0 = lax.min(cur0 + 1, jnp.int32(_NBLK - 1))
    stage(nxt0, 1, psem)

    def group_body(g, carry):
        cur, s, pf = carry
        qg = q_v[pl.ds(g * 16, 16)]
        blocks = qg >> 9
        cols = qg & (_BW - 1)
        rowbase = ((lax.rem(g, _GPF) * 16) + lax.iota(jnp.int32, 16)) * _PITCH

        def run_body(state):
            cur, s, pf, rem = state
            t = lax.reduce_min(jnp.where(rem, blocks, _BIG), (0,))

            need = t != cur

            def advance(args):
                cur, s, pf = args
                stage_wait(pf, 1 - s, psem)
                s2 = 1 - s

                @pl.when(pf != t)
                def _():
                    stage(t, s2, ssem)
                    stage_wait(t, s2, ssem)

                nxt = lax.min(t + 1, jnp.int32(_NBLK - 1))
                stage(nxt, 1 - s2, psem)
                return t, s2, nxt

            cur, s, pf = lax.cond(need, advance, lambda a: a, (cur, s, pf))

            m = rem & (blocks == t)
            partial = (t - (t // _BPF) * _BPF) == (_BPF - 1)
            m_tail = m & partial & (cols >= 128)
            m_main = m & jnp.logical_not(m_tail)
            sidx = jnp.full((16,), s, dtype=jnp.int32)
            for e in range(_EMB):
                eidx = jnp.full((16,), e, dtype=jnp.int32)
                vals = plsc.load_gather(blk_v, [sidx, eidx, cols])
                plsc.store_scatter(rows_v, [rowbase + e], vals, mask=m_main)

            @pl.when(jnp.any(m_tail))
            def _():
                cols_t = jnp.where(m_tail, cols - 128, 0)
                for e in range(_EMB):
                    eidx = jnp.full((16,), e, dtype=jnp.int32)
                    vals = plsc.load_gather(pblk_v, [sidx, eidx, cols_t])
                    plsc.store_scatter(rows_v, [rowbase + e], vals, mask=m_tail)

            return cur, s, pf, rem & jnp.logical_not(m)

        def run_cond(state):
            return jnp.any(state[3])

        cur, s, pf, _ = lax.while_loop(
            run_cond, run_body,
            (cur, s, pf, jnp.ones((16,), dtype=jnp.bool_)))

        @pl.when(lax.rem(g, _GPF) == _GPF - 1)
        def _():
            pltpu.async_copy(
                rows_v,
                out_hbm.at[pl.ds((base + (g - (_GPF - 1)) * 16) * _PITCH,
                                 _GPF * 16 * _PITCH)],
                fsem,
            ).wait()

        return cur, s, pf

    cur, s, pf = lax.fori_loop(0, _NGRP, group_body, (cur0, 0, nxt0))
    stage_wait(pf, 1 - s, psem)


_CH = 128
_NCH = _PPW // _CH


_PADCH = _CH * _PITCH


@functools.partial(
    pl.kernel,
    out_type=jax.ShapeDtypeStruct((_TOTAL_ROWS, _EMB), jnp.float32),
    mesh=_sc_mesh,
    scratch_types=[
        pltpu.VMEM((_NCH, _CH), jnp.int32),
        pltpu.VMEM((_PADCH,), jnp.float32),
        pltpu.VMEM((_PADCH,), jnp.float32),
        pltpu.VMEM((_CH, _EMB), jnp.float32),
        pltpu.SemaphoreType.DMA,
        pltpu.SemaphoreType.DMA,
        pltpu.SemaphoreType.DMA,
    ],
    compiler_params=pltpu.CompilerParams(use_tc_tiling_on_sc=False),
)
def _sc_unpermute(src_hbm, rid_hbm, out_hbm, rid_v, pad0, pad1, rows_v,
                  s0, s1, wsem):
    wid = lax.axis_index("s") * _NC + lax.axis_index("c")
    pltpu.sync_copy(rid_hbm.at[wid], rid_v)
    base = wid * _NCH

    def load(c, buf, sem):
        pltpu.async_copy(src_hbm.at[pl.ds((base + c) * _PADCH, _PADCH)],
                         buf, sem)

    def load_wait(c, buf, sem):
        pltpu.make_async_copy(
            src_hbm.at[pl.ds((base + c) * _PADCH, _PADCH)], buf, sem).wait()

    def compact_scatter(c, buf):
        for k in range(_CH):
            for mm in range(_EMB // 16):
                rows_v[k, pl.ds(mm * 16, 16)] = buf[pl.ds(k * _PITCH + mm * 16, 16)]
        pltpu.async_copy(rows_v, out_hbm.at[rid_v.at[c]], wsem).wait()

    load(0, pad0, s0)

    def body(j, carry):
        c0 = 2 * j
        c1 = 2 * j + 1
        load(c1, pad1, s1)
        load_wait(c0, pad0, s0)
        compact_scatter(c0, pad0)

        @pl.when(c1 + 1 < _NCH)
        def _():
            load(c1 + 1, pad0, s0)

        load_wait(c1, pad1, s1)
        compact_scatter(c1, pad1)
        return carry

    lax.fori_loop(0, _NCH // 2, body, 0)


_BM = 1024


def _mlp_body(x_ref, w1_ref, b1_ref, w2_ref, b2_ref, w3_ref, b3_ref, o_ref):
    h = jnp.dot(x_ref[...].astype(jnp.bfloat16), w1_ref[...].astype(jnp.bfloat16),
                preferred_element_type=jnp.float32)
    h = jnp.maximum(h + b1_ref[...], 0.0).astype(jnp.bfloat16)
    h = jnp.dot(h, w2_ref[...].astype(jnp.bfloat16),
                preferred_element_type=jnp.float32)
    h = jnp.maximum(h + b2_ref[...], 0.0).astype(jnp.bfloat16)
    o_ref[...] = jnp.dot(h, w3_ref[...].astype(jnp.bfloat16),
                         preferred_element_type=jnp.float32) + b3_ref[...]


def _tc_mlp(x, W1, b1, W2, b2, W3, b3):
    nb = _BATCH // _BM
    return pl.pallas_call(
        _mlp_body,
        grid=(nb,),
        in_specs=[
            pl.BlockSpec((_BM, _CAT), lambda i: (i, 0)),
            pl.BlockSpec((_CAT, _H0), lambda i: (0, 0)),
            pl.BlockSpec((1, _H0), lambda i: (0, 0)),
            pl.BlockSpec((_H0, _H1), lambda i: (0, 0)),
            pl.BlockSpec((1, _H1), lambda i: (0, 0)),
            pl.BlockSpec((_H1, _OUT), lambda i: (0, 0)),
            pl.BlockSpec((1, _OUT), lambda i: (0, 0)),
        ],
        out_specs=pl.BlockSpec((_BM, _OUT), lambda i: (i, 0)),
        out_shape=jax.ShapeDtypeStruct((_BATCH, _OUT), jnp.float32),
    )(x, W1, b1.reshape(1, _H0), W2, b2.reshape(1, _H1), W3, b3.reshape(1, _OUT))


def _tower_gather(emb, field_idx):
    off = jnp.arange(_NUM_FIELDS, dtype=jnp.int32) * _QSTRIDE
    q = (field_idx.astype(jnp.int32) + off[None, :]).reshape(-1)
    rid = lax.iota(jnp.int32, _TOTAL_ROWS)
    q_sorted, rid_sorted = lax.sort([q, rid], num_keys=1)
    tt = jnp.transpose(emb, (0, 2, 1))
    flat = _sc_scan_gather(tt, q_sorted)
    out = _sc_unpermute(flat, rid_sorted.reshape(_NW, _NCH, _CH))
    return out.reshape(_BATCH, _CAT)


def kernel(user_input, item_input, user_emb, item_emb, W_user, b_user,
           W_item, b_item, W2, b2, W3, b3):
    u_cat = _tower_gather(user_emb, user_input)
    out1 = _tc_mlp(u_cat, W_user, b_user, W2, b2, W3, b3)
    v_cat = _tower_gather(item_emb, item_input)
    out2 = _tc_mlp(v_cat, W_item, b_item, W2, b2, W3, b3)
    return (out1, out2)
